# Initial kernel scaffold; baseline (speedup 1.0000x reference)
import jax
import jax.numpy as jnp
from jax.experimental import pallas as pl


def kernel(x):
    raise NotImplementedError("write your pallas kernel here")


# speedup vs baseline: 62.8449x; 62.8449x over previous
import functools

import jax
import jax.numpy as jnp
from jax.experimental import pallas as pl
from jax.experimental.pallas import tpu as pltpu

_MASK_RATIO = 0.25


def _tc_body(x_ref, o_ref, *, k):
    v = x_ref[...]
    bits = jax.lax.bitcast_convert_type(v, jnp.int32) & jnp.int32(0x7FFFFFFF)

    rows = v.shape[0]
    t0 = jnp.zeros((rows, 1), jnp.int32)

    def step(i, t):
        cand = t | jax.lax.shift_left(jnp.int32(1), jnp.int32(30) - i)
        cnt = jnp.sum((bits >= cand).astype(jnp.int32), axis=1, keepdims=True)
        return jnp.where(cnt >= k, cand, t)

    t = jax.lax.fori_loop(0, 31, step, t0)
    o_ref[...] = jnp.where(bits >= t, jnp.float32(1.0), jnp.float32(0.0))


def kernel(x):
    B, N, C, L = x.shape
    k = int(L * _MASK_RATIO)
    M = B * N * C
    rows_per_block = 256
    while M % rows_per_block:
        rows_per_block //= 2
    xf = x.reshape(M, L)

    out = pl.pallas_call(
        functools.partial(_tc_body, k=k),
        grid=(M // rows_per_block,),
        in_specs=[pl.BlockSpec((rows_per_block, L), lambda i: (i, 0))],
        out_specs=pl.BlockSpec((rows_per_block, L), lambda i: (i, 0)),
        out_shape=jax.ShapeDtypeStruct((M, L), jnp.float32),
    )(xf)
    return out.reshape(B, N, C, L)
